# baseline (speedup 1.0000x reference)
import functools

import jax
import jax.numpy as jnp
from jax import lax
from jax.experimental import pallas as pl
from jax.experimental.pallas import tpu as pltpu
from jax.experimental.pallas import tpu_sc as plsc

NUM_CODES = 1024
DIM = 64
N_ROWS = 64 * 576
BLOCK_ROWS = 512
N_CHUNKS = 4
CHUNK_ROWS = N_ROWS // N_CHUNKS
CHUNK_BLOCKS = CHUNK_ROWS // BLOCK_ROWS
COMMIT = 0.25


def _vq_block(x_ref, e_ref, idx_ref, counts_ref, msum_ref,
              esq_ref, iotaf_ref):
    i = pl.program_id(0)

    @pl.when(i == 0)
    def _init():
        counts_ref[...] = jnp.zeros_like(counts_ref)
        msum_ref[...] = jnp.zeros_like(msum_ref)
        e0 = e_ref[...]
        esq_ref[...] = jnp.sum(e0 * e0, axis=1)[None, :]
        iotaf_ref[...] = lax.broadcasted_iota(
            jnp.int32, (1, NUM_CODES), 1).astype(jnp.float32)

    x = x_ref[...]
    x_sq = jnp.sum(x * x, axis=1, keepdims=True)
    mm = lax.dot_general(x, e_ref[...], (((1,), (1,)), ((), ())),
                         preferred_element_type=jnp.float32)
    dist = x_sq + esq_ref[...] - 2.0 * mm
    m = jnp.min(dist, axis=1, keepdims=True)
    hit = dist == m
    enc = jnp.min(jnp.where(hit, iotaf_ref[...], float(NUM_CODES)),
                  axis=1).astype(jnp.int32)
    idx_ref[...] = enc

    counts_ref[...] += jnp.sum(jnp.where(hit, 1.0, 0.0), axis=0, keepdims=True)
    msum_ref[...] += jnp.full((1, 1), 1.0) * jnp.sum(m)


@functools.cache
def _make_vq_chunk(chunk):
    return pl.pallas_call(
        _vq_block,
        grid=(CHUNK_BLOCKS,),
        in_specs=[
            pl.BlockSpec((BLOCK_ROWS, DIM),
                         lambda i, c=chunk: (c * CHUNK_BLOCKS + i, 0)),
            pl.BlockSpec((NUM_CODES, DIM), lambda i: (0, 0)),
        ],
        out_specs=[
            pl.BlockSpec((BLOCK_ROWS,), lambda i: (i,)),
            pl.BlockSpec((1, NUM_CODES), lambda i: (0, 0)),
            pl.BlockSpec((1, 1), lambda i: (0, 0)),
        ],
        out_shape=[
            jax.ShapeDtypeStruct((CHUNK_ROWS,), jnp.int32),
            jax.ShapeDtypeStruct((1, NUM_CODES), jnp.float32),
            jax.ShapeDtypeStruct((1, 1), jnp.float32),
        ],
        scratch_shapes=[
            pltpu.VMEM((1, NUM_CODES), jnp.float32),
            pltpu.VMEM((1, NUM_CODES), jnp.float32),
        ],
    )


def _combine_block(counts_ref, msum_ref, loss_ref, perp_ref):
    msum = jnp.sum(msum_ref[...])
    loss_ref[...] = jnp.full((1, 1), COMMIT / (N_ROWS * DIM)) * msum
    counts = jnp.sum(counts_ref[...], axis=0, keepdims=True)
    p = counts * (1.0 / N_ROWS)
    ent = jnp.sum(p * jnp.log(p + 1e-10))
    perp_ref[...] = jnp.exp(jnp.full((1, 1), -ent))


_combine_call = pl.pallas_call(
    _combine_block,
    out_shape=[
        jax.ShapeDtypeStruct((1, 1), jnp.float32),
        jax.ShapeDtypeStruct((1, 1), jnp.float32),
    ],
)


_SC_CORES = 2
_SC_SUBCORES = 16
_NW = _SC_CORES * _SC_SUBCORES
_ROWS_PER_W = CHUNK_ROWS // _NW


def _sc_gather_body(table_hbm, idx_hbm, out_hbm, idx_v, rows_v, sem):
    wid = lax.axis_index("s") * _SC_CORES + lax.axis_index("c")
    base = wid * _ROWS_PER_W
    pltpu.sync_copy(idx_hbm.at[pl.ds(base, _ROWS_PER_W)], idx_v)
    pltpu.async_copy(table_hbm.at[idx_v], rows_v, sem).wait()
    pltpu.sync_copy(rows_v, out_hbm.at[pl.ds(base, _ROWS_PER_W)])


@functools.cache
def _make_sc_gather():
    return pl.kernel(
        _sc_gather_body,
        out_type=jax.ShapeDtypeStruct((CHUNK_ROWS, DIM), jnp.float32),
        mesh=plsc.VectorSubcoreMesh(core_axis_name="c", subcore_axis_name="s"),
        scratch_types=[
            pltpu.VMEM((_ROWS_PER_W,), jnp.int32),
            pltpu.VMEM((_ROWS_PER_W, DIM), jnp.float32),
            pltpu.SemaphoreType.DMA,
        ],
        compiler_params=pltpu.CompilerParams(use_tc_tiling_on_sc=False),
    )


@jax.jit
def kernel(inputs, embedding):
    shape = inputs.shape
    flat = inputs.reshape(-1, DIM)
    encs, counts, msums, quants = [], [], [], []
    gather = _make_sc_gather()
    for c in range(N_CHUNKS):
        enc_c, counts_c, msum_c = _make_vq_chunk(c)(flat, embedding)
        encs.append(enc_c)
        counts.append(counts_c)
        msums.append(msum_c)
        quants.append(gather(embedding, enc_c))
    loss, perp = _combine_call(jnp.concatenate(counts, axis=0),
                               jnp.concatenate(msums, axis=0))
    enc = jnp.concatenate(encs)
    quantized = jnp.concatenate(quants, axis=0)
    return (embedding,
            loss[0, 0],
            quantized.reshape(shape),
            perp[0, 0],
            enc.reshape(shape[0], shape[1]))


# speedup vs baseline: 1.1085x; 1.1085x over previous
import functools

import jax
import jax.numpy as jnp
from jax import lax
from jax.experimental import pallas as pl
from jax.experimental.pallas import tpu as pltpu
from jax.experimental.pallas import tpu_sc as plsc

NUM_CODES = 1024
DIM = 64
N_ROWS = 64 * 576
BLOCK_ROWS = 512
N_BLOCKS = N_ROWS // BLOCK_ROWS
COMMIT = 0.25
HROWS, HCOLS = 16, NUM_CODES // 16


def _vq_block(x_ref, e_ref, idx_ref, msum_ref, esq_ref, iotaf_ref):
    i = pl.program_id(0)

    @pl.when(i == 0)
    def _init():
        msum_ref[...] = jnp.zeros_like(msum_ref)
        e0 = e_ref[...]
        esq_ref[...] = jnp.sum(e0 * e0, axis=1)[None, :]
        iotaf_ref[...] = lax.broadcasted_iota(
            jnp.int32, (1, NUM_CODES), 1).astype(jnp.float32)

    x = x_ref[...]
    x_sq = jnp.sum(x * x, axis=1, keepdims=True)
    mm = lax.dot_general(x, e_ref[...], (((1,), (1,)), ((), ())),
                         preferred_element_type=jnp.float32)
    dist = x_sq + esq_ref[...] - 2.0 * mm
    m = jnp.min(dist, axis=1, keepdims=True)
    enc = jnp.min(jnp.where(dist == m, iotaf_ref[...], float(NUM_CODES)),
                  axis=1).astype(jnp.int32)
    idx_ref[...] = enc
    msum_ref[...] += jnp.full((1, 1), 1.0) * jnp.sum(m)


_vq_call = pl.pallas_call(
    _vq_block,
    grid=(N_BLOCKS,),
    in_specs=[
        pl.BlockSpec((BLOCK_ROWS, DIM), lambda i: (i, 0)),
        pl.BlockSpec((NUM_CODES, DIM), lambda i: (0, 0)),
    ],
    out_specs=[
        pl.BlockSpec((BLOCK_ROWS,), lambda i: (i,)),
        pl.BlockSpec((1, 1), lambda i: (0, 0)),
    ],
    out_shape=[
        jax.ShapeDtypeStruct((N_ROWS,), jnp.int32),
        jax.ShapeDtypeStruct((1, 1), jnp.float32),
    ],
    scratch_shapes=[
        pltpu.VMEM((1, NUM_CODES), jnp.float32),
        pltpu.VMEM((1, NUM_CODES), jnp.float32),
    ],
)


def _combine_block(counts_ref, msum_ref, loss_ref, perp_ref):
    loss_ref[...] = jnp.full((1, 1), COMMIT / (N_ROWS * DIM)) * msum_ref[0, 0]
    counts = counts_ref[0] + counts_ref[1]
    p = counts * (1.0 / N_ROWS)
    ent = jnp.sum(p * jnp.log(p + 1e-10))
    perp_ref[...] = jnp.exp(jnp.full((1, 1), -ent))


_combine_call = pl.pallas_call(
    _combine_block,
    out_shape=[
        jax.ShapeDtypeStruct((1, 1), jnp.float32),
        jax.ShapeDtypeStruct((1, 1), jnp.float32),
    ],
)


_SC_CORES = 2
_SC_SUBCORES = 16
_NW = _SC_CORES * _SC_SUBCORES
_ROWS_PER_W = N_ROWS // _NW
_VECS_PER_W = _ROWS_PER_W // 16


def _sc_gather_body(table_hbm, idx_hbm, out_hbm, counts_hbm,
                    idx_v, rows_v, hist_v, zero_v, shared_hist, sem):
    sid = lax.axis_index("s")
    cid = lax.axis_index("c")
    wid = sid * _SC_CORES + cid
    base = wid * _ROWS_PER_W
    pltpu.sync_copy(idx_hbm.at[pl.ds(base, _ROWS_PER_W)], idx_v)
    gather = pltpu.async_copy(table_hbm.at[idx_v], rows_v, sem)

    zeros16 = jnp.zeros((16,), jnp.float32)
    for r in range(HROWS):
        for c in range(HCOLS // 16):
            hist_v[r, pl.ds(c * 16, 16)] = zeros16
            zero_v[r, pl.ds(c * 16, 16)] = zeros16
    ones16 = jnp.ones((16,), jnp.float32)

    def hist_step(k, carry):
        iv = idx_v[pl.ds(k * 16, 16)]
        plsc.addupdate_scatter(
            hist_v, [iv >> 6, iv & (HCOLS - 1)], ones16)
        return carry

    lax.fori_loop(0, _VECS_PER_W, hist_step, 0, unroll=8)

    row_ids = lax.iota(jnp.int32, 16)

    @pl.when(sid == 0)
    def _zero_shared():
        pltpu.sync_copy(zero_v, shared_hist)

    plsc.subcore_barrier()
    pltpu.sync_copy(hist_v, shared_hist.at[row_ids], add=True)
    plsc.subcore_barrier()

    @pl.when(sid == 0)
    def _emit_counts():
        pltpu.sync_copy(shared_hist, hist_v)
        pltpu.sync_copy(hist_v, counts_hbm.at[cid])

    gather.wait()
    pltpu.sync_copy(rows_v, out_hbm.at[pl.ds(base, _ROWS_PER_W)])


@functools.cache
def _make_sc_gather():
    return pl.kernel(
        _sc_gather_body,
        out_type=(
            jax.ShapeDtypeStruct((N_ROWS, DIM), jnp.float32),
            jax.ShapeDtypeStruct((_SC_CORES, HROWS, HCOLS), jnp.float32),
        ),
        mesh=plsc.VectorSubcoreMesh(core_axis_name="c", subcore_axis_name="s"),
        scratch_types=[
            pltpu.VMEM((_ROWS_PER_W,), jnp.int32),
            pltpu.VMEM((_ROWS_PER_W, DIM), jnp.float32),
            pltpu.VMEM((HROWS, HCOLS), jnp.float32),
            pltpu.VMEM((HROWS, HCOLS), jnp.float32),
            pltpu.VMEM_SHARED((HROWS, HCOLS), jnp.float32),
            pltpu.SemaphoreType.DMA,
        ],
        compiler_params=pltpu.CompilerParams(use_tc_tiling_on_sc=False,
                                             needs_layout_passes=False),
    )


@jax.jit
def kernel(inputs, embedding):
    shape = inputs.shape
    flat = inputs.reshape(-1, DIM)
    enc, msum = _vq_call(flat, embedding)
    quantized, counts = _make_sc_gather()(embedding, enc)
    loss, perp = _combine_call(counts, msum)
    return (embedding,
            loss[0, 0],
            quantized.reshape(shape),
            perp[0, 0],
            enc.reshape(shape[0], shape[1]))
